# baseline (speedup 1.0000x reference)
import jax
import jax.numpy as jnp
import numpy as np
from jax import lax
from jax.experimental import pallas as pl
from jax.experimental.pallas import tpu as pltpu

LANES = 128
ROWS_PER_BLOCK = 1024

_M4 = np.uint32(0x0F0F0F0F)
_M2 = np.uint32(0x33333333)
_M1 = np.uint32(0x55555555)
_ALL1 = np.uint32(0xFFFFFFFF)


def _brev32(v):
    v = ((v & _M4) << 4) | ((v >> 4) & _M4)
    v = ((v & _M2) << 2) | ((v >> 2) & _M2)
    v = ((v & _M1) << 1) | ((v >> 1) & _M1)
    return v


def _bsadd_block(a_ref, b_ref, o_ref, carry_ref):
    @pl.when(pl.program_id(0) == 0)
    def _():
        carry_ref[0] = 0

    a = _brev32(a_ref[...])
    b = _brev32(b_ref[...])
    s = a + b
    g = (s < a).astype(jnp.int32)
    p = (s == _ALL1)
    f = jnp.where(p, 2, g)

    R, C = f.shape
    two = jnp.int32(2)

    k = 1
    while k < C:
        shifted = jnp.concatenate(
            [jnp.full((R, k), two, jnp.int32), f[:, : C - k]], axis=1)
        f = jnp.where(f == two, shifted, f)
        k *= 2

    agg = f[:, C - 1:C]
    k = 1
    while k < R:
        shifted = jnp.concatenate(
            [jnp.full((k, 1), two, jnp.int32), agg[: R - k]], axis=0)
        agg = jnp.where(agg == two, shifted, agg)
        k *= 2

    carry_in = carry_ref[0]
    row_excl = jnp.concatenate(
        [jnp.full((1, 1), two, jnp.int32), agg[: R - 1]], axis=0)
    row_pref = jnp.where(row_excl == two, carry_in, row_excl)

    e = jnp.concatenate(
        [jnp.full((R, 1), two, jnp.int32), f[:, : C - 1]], axis=1)
    cin = jnp.where(e == two, row_pref, e)

    o_ref[...] = _brev32(s + cin.astype(jnp.uint32))

    block_fold = agg[R - 1, 0]
    carry_ref[0] = jnp.where(block_fold == two, carry_in, block_fold)


@jax.jit
def kernel(a, b):
    n = a.shape[0]
    aw = lax.bitcast_convert_type(a.reshape(n // 4, 4), jnp.uint32)
    bw = lax.bitcast_convert_type(b.reshape(n // 4, 4), jnp.uint32)
    rows = n // 4 // LANES
    aw = aw.reshape(rows, LANES)
    bw = bw.reshape(rows, LANES)
    rpb = min(ROWS_PER_BLOCK, rows)
    grid = rows // rpb

    out = pl.pallas_call(
        _bsadd_block,
        grid=(grid,),
        in_specs=[
            pl.BlockSpec((rpb, LANES), lambda i: (i, 0)),
            pl.BlockSpec((rpb, LANES), lambda i: (i, 0)),
        ],
        out_specs=pl.BlockSpec((rpb, LANES), lambda i: (i, 0)),
        out_shape=jax.ShapeDtypeStruct((rows, LANES), jnp.uint32),
        scratch_shapes=[pltpu.SMEM((1,), jnp.int32)],
        compiler_params=pltpu.CompilerParams(
            dimension_semantics=("arbitrary",)),
    )(aw, bw)

    return lax.bitcast_convert_type(out.reshape(n // 4), jnp.uint8).reshape(n)


# speedup vs baseline: 16.3803x; 16.3803x over previous
import jax
import jax.numpy as jnp
import numpy as np
from jax import lax
from jax.experimental import pallas as pl
from jax.experimental.pallas import tpu as pltpu

LANES = 128
ROWS_PER_BLOCK = 1024


def _brev8(v):
    v = ((v & 0x0F) << 4) | (v >> 4)
    v = ((v & 0x33) << 2) | ((v >> 2) & 0x33)
    v = ((v & 0x55) << 1) | ((v >> 1) & 0x55)
    return v


def _bsadd_block(a_ref, b_ref, o_ref, carry_ref):
    @pl.when(pl.program_id(0) == 0)
    def _():
        carry_ref[0] = 0

    a = _brev8(a_ref[...].astype(jnp.int32))
    b = _brev8(b_ref[...].astype(jnp.int32))
    c = a + b
    g = c >> 8
    c = c & 0xFF
    f = jnp.where(c == 0xFF, 2, g)

    R, C = f.shape
    two = jnp.int32(2)

    k = 1
    while k < C:
        shifted = jnp.concatenate(
            [jnp.full((R, k), two, jnp.int32), f[:, : C - k]], axis=1)
        f = jnp.where(f == two, shifted, f)
        k *= 2

    agg = f[:, C - 1:C]
    k = 1
    while k < R:
        shifted = jnp.concatenate(
            [jnp.full((k, 1), two, jnp.int32), agg[: R - k]], axis=0)
        agg = jnp.where(agg == two, shifted, agg)
        k *= 2

    carry_in = carry_ref[0]
    row_excl = jnp.concatenate(
        [jnp.full((1, 1), two, jnp.int32), agg[: R - 1]], axis=0)
    row_pref = jnp.where(row_excl == two, carry_in, row_excl)

    e = jnp.concatenate(
        [jnp.full((R, 1), two, jnp.int32), f[:, : C - 1]], axis=1)
    cin = jnp.where(e == two, row_pref, e)

    o_ref[...] = _brev8((c + cin) & 0xFF).astype(jnp.uint8)

    block_fold = agg[R - 1, 0]
    carry_ref[0] = jnp.where(block_fold == two, carry_in, block_fold)


@jax.jit
def kernel(a, b):
    n = a.shape[0]
    rows = n // LANES
    a2 = a.reshape(rows, LANES)
    b2 = b.reshape(rows, LANES)
    rpb = min(ROWS_PER_BLOCK, rows)
    grid = rows // rpb

    out = pl.pallas_call(
        _bsadd_block,
        grid=(grid,),
        in_specs=[
            pl.BlockSpec((rpb, LANES), lambda i: (i, 0)),
            pl.BlockSpec((rpb, LANES), lambda i: (i, 0)),
        ],
        out_specs=pl.BlockSpec((rpb, LANES), lambda i: (i, 0)),
        out_shape=jax.ShapeDtypeStruct((rows, LANES), jnp.uint8),
        scratch_shapes=[pltpu.SMEM((1,), jnp.int32)],
        compiler_params=pltpu.CompilerParams(
            dimension_semantics=("arbitrary",)),
    )(a2, b2)

    return out.reshape(n)


# speedup vs baseline: 17.3439x; 1.0588x over previous
import jax
import jax.numpy as jnp
import numpy as np
from jax import lax
from jax.experimental import pallas as pl
from jax.experimental.pallas import tpu as pltpu

LANES = 128
ROWS_PER_BLOCK = 1024


def _brev8(v):
    v = ((v & 0x0F) << 4) | (v >> 4)
    v = ((v & 0x33) << 2) | ((v >> 2) & 0x33)
    v = ((v & 0x55) << 1) | ((v >> 1) & 0x55)
    return v


def _bsadd_block(a_ref, b_ref, o_ref, carry_ref):
    @pl.when(pl.program_id(0) == 0)
    def _():
        carry_ref[0] = 0

    a = _brev8(a_ref[...].astype(jnp.int32))
    b = _brev8(b_ref[...].astype(jnp.int32))
    c = a + b
    g = c >> 8
    c = c & 0xFF
    p = (c == 0xFF)

    R, C = c.shape
    H = C // 16
    two = jnp.int32(2)

    ji = lax.broadcasted_iota(jnp.int32, (C, H), 0)
    hi = lax.broadcasted_iota(jnp.int32, (C, H), 1)
    w_pack = jnp.where(ji // 16 == hi, 1 << (ji % 16), 0).astype(jnp.float32)
    gm = jnp.dot(g.astype(jnp.float32), w_pack,
                 preferred_element_type=jnp.float32).astype(jnp.int32)
    pm = jnp.dot(p.astype(jnp.float32), w_pack,
                 preferred_element_type=jnp.float32).astype(jnp.int32)

    pg = pm | gm
    fh = jnp.where(pm == 0xFFFF, 2, (gm + pg) >> 16)

    f = fh
    k = 1
    while k < H:
        shifted = jnp.concatenate(
            [jnp.full((R, k), two, jnp.int32), f[:, : H - k]], axis=1)
        f = jnp.where(f == two, shifted, f)
        k *= 2

    agg = f[:, H - 1:H]
    k = 1
    while k < R:
        shifted = jnp.concatenate(
            [jnp.full((k, 1), two, jnp.int32), agg[: R - k]], axis=0)
        agg = jnp.where(agg == two, shifted, agg)
        k *= 2

    carry_in = carry_ref[0]
    row_excl = jnp.concatenate(
        [jnp.full((1, 1), two, jnp.int32), agg[: R - 1]], axis=0)
    row_pref = jnp.where(row_excl == two, carry_in, row_excl)

    e = jnp.concatenate(
        [jnp.full((R, 1), two, jnp.int32), f[:, : H - 1]], axis=1)
    ch = jnp.where(e == two, row_pref, e)

    carries = ((gm + pg + ch) ^ gm ^ pg) & 0xFFFF

    hj = lax.broadcasted_iota(jnp.int32, (H, C), 0)
    jj = lax.broadcasted_iota(jnp.int32, (H, C), 1)
    e_sel = (jj // 16 == hj).astype(jnp.float32)
    v = jnp.dot(carries.astype(jnp.float32), e_sel,
                preferred_element_type=jnp.float32).astype(jnp.int32)
    lane = lax.broadcasted_iota(jnp.int32, (R, C), 1)
    cin = (v >> (lane % 16)) & 1

    o_ref[...] = _brev8((c + cin) & 0xFF).astype(jnp.uint8)

    block_fold = agg[R - 1, 0]
    carry_ref[0] = jnp.where(block_fold == two, carry_in, block_fold)


@jax.jit
def kernel(a, b):
    n = a.shape[0]
    rows = n // LANES
    a2 = a.reshape(rows, LANES)
    b2 = b.reshape(rows, LANES)
    rpb = min(ROWS_PER_BLOCK, rows)
    grid = rows // rpb

    out = pl.pallas_call(
        _bsadd_block,
        grid=(grid,),
        in_specs=[
            pl.BlockSpec((rpb, LANES), lambda i: (i, 0)),
            pl.BlockSpec((rpb, LANES), lambda i: (i, 0)),
        ],
        out_specs=pl.BlockSpec((rpb, LANES), lambda i: (i, 0)),
        out_shape=jax.ShapeDtypeStruct((rows, LANES), jnp.uint8),
        scratch_shapes=[pltpu.SMEM((1,), jnp.int32)],
        compiler_params=pltpu.CompilerParams(
            dimension_semantics=("arbitrary",)),
    )(a2, b2)

    return out.reshape(n)
